# baseline (speedup 1.0000x reference)
import functools

import jax
import jax.numpy as jnp
from jax import lax
from jax.experimental import pallas as pl
from jax.experimental.pallas import tpu as pltpu
from jax.experimental.pallas import tpu_sc as plsc

B, S, L, MM, H = 8, 3, 2048, 1024, 1024
NC, NS = 2, 16
NW = NC * NS
NSEG = B * S
NQ = 4
NTASK = NSEG * NQ
TPW = NTASK // NW
QROWS = L // NQ
CH = 32
NCH = QROWS // CH
COL_U = 4
VL = 16


def _sc_partial_sums(x2, mf96):
    mesh = plsc.VectorSubcoreMesh(
        core_axis_name="c", subcore_axis_name="s", num_cores=NC, num_subcores=NS
    )

    @functools.partial(
        pl.kernel,
        out_type=jax.ShapeDtypeStruct((NQ, NSEG, MM), jnp.float32),
        mesh=mesh,
        scratch_types=[
            pltpu.VMEM((CH, MM), jnp.float32),
            pltpu.VMEM((QROWS,), jnp.float32),
            pltpu.VMEM((MM,), jnp.float32),
        ],
    )
    def k(x_hbm, mf_hbm, out_hbm, buf, mfb, acc):
        wid = lax.axis_index("s") * NC + lax.axis_index("c")

        def task_body(ti, _):
            t = wid * TPW + ti
            seg = t // NQ
            q = t - seg * NQ
            row0 = t * QROWS
            pltpu.sync_copy(mf_hbm.at[t], mfb)

            def zero_body(c, _):
                acc[pl.ds(c * VL, VL)] = jnp.zeros((VL,), jnp.float32)
                return 0

            lax.fori_loop(0, MM // VL, zero_body, 0)

            def chunk_body(ch, _):
                pltpu.sync_copy(x_hbm.at[pl.ds(row0 + ch * CH, CH)], buf)

                def row_body(rg, _):
                    r0 = rg * VL
                    mvec = mfb[pl.ds(ch * CH + r0, VL)]
                    ms = [mvec[j] for j in range(VL)]

                    def col_body(cb, _):
                        for cs in range(COL_U):
                            o = cb * (COL_U * VL) + cs * VL
                            v = acc[pl.ds(o, VL)]
                            for j in range(VL):
                                v = v + buf[r0 + j, pl.ds(o, VL)] * ms[j]
                            acc[pl.ds(o, VL)] = v
                        return 0

                    lax.fori_loop(0, MM // (COL_U * VL), col_body, 0)
                    return 0

                lax.fori_loop(0, CH // VL, row_body, 0)
                return 0

            lax.fori_loop(0, NCH, chunk_body, 0)
            pltpu.sync_copy(acc, out_hbm.at[q, seg])
            return 0

        lax.fori_loop(0, TPW, task_body, 0)

    return k(x2, mf96)


def _tc_finish(part, mf24, gf_col, w_proj, b_proj, wc24, bc24):
    def body(part_ref, mf_ref, gf_ref, wp_ref, bp_ref, wc_ref, bc_ref, out_ref):
        seg_sum = part_ref[0] + part_ref[1] + part_ref[2] + part_ref[3]
        den = jnp.sum(mf_ref[:], axis=1, keepdims=True)
        logits = lax.dot_general(
            wc_ref[:], gf_ref[:], (((1,), (0,)), ((), ())),
            precision=lax.Precision.HIGHEST,
        ) + bc_ref[:]
        m = jnp.max(logits, axis=0, keepdims=True)
        e = jnp.exp(logits - m)
        s = jnp.sum(e, axis=0, keepdims=True) / B
        w24 = e / s
        scaled = seg_sum * (w24 / den)
        ri = lax.broadcasted_iota(jnp.int32, (B, NSEG), 0)
        cj = lax.broadcasted_iota(jnp.int32, (B, NSEG), 1)
        sel = jnp.where((cj >= S * ri) & (cj < S * ri + S), 1.0, 0.0)
        xw = lax.dot_general(
            sel, scaled, (((1,), (0,)), ((), ())),
            precision=lax.Precision.HIGHEST,
        )
        out_ref[:] = lax.dot_general(
            xw, wp_ref[:], (((1,), (1,)), ((), ())),
            precision=lax.Precision.HIGHEST,
        ) + bp_ref[:]

    return pl.pallas_call(
        body, out_shape=jax.ShapeDtypeStruct((B, H), jnp.float32)
    )(part, mf24, gf_col, w_proj, b_proj, wc24, bc24)


@jax.jit
def kernel(graph_feature, x_tensors, x_mask, W_proj, b_proj, W_comb, b_comb):
    mf = (~x_mask).astype(jnp.float32)
    x2 = x_tensors.reshape(B * S * L, MM)
    mf96 = mf.reshape(NTASK, QROWS)
    part = _sc_partial_sums(x2, mf96)
    mf24 = mf.reshape(NSEG, L)
    gf_col = graph_feature.reshape(MM, 1)
    wc24 = jnp.tile(W_comb, (B, 1))
    bc24 = jnp.tile(b_comb, (B,)).reshape(NSEG, 1)
    bp = b_proj.reshape(1, H)
    return _tc_finish(part, mf24, gf_col, W_proj, bp, wc24, bc24)


# speedup vs baseline: 1.4426x; 1.4426x over previous
import functools

import jax
import jax.numpy as jnp
from jax import lax
from jax.experimental import pallas as pl
from jax.experimental.pallas import tpu as pltpu
from jax.experimental.pallas import tpu_sc as plsc

B, S, L, MM, H = 8, 3, 2048, 1024, 1024
NC, NS = 2, 16
NW = NC * NS
NSEG = B * S
NQ = 4
NTASK = NSEG * NQ
TPW = NTASK // NW
QROWS = L // NQ
CH = 32
NCH = QROWS // CH
COL_U = 4
VL = 16


def _sc_partial_sums(x2, mf96):
    mesh = plsc.VectorSubcoreMesh(
        core_axis_name="c", subcore_axis_name="s", num_cores=NC, num_subcores=NS
    )

    @functools.partial(
        pl.kernel,
        out_type=jax.ShapeDtypeStruct((NQ, NSEG, MM), jnp.float32),
        mesh=mesh,
        scratch_types=[
            pltpu.VMEM((CH, MM), jnp.float32),
            pltpu.VMEM((CH, MM), jnp.float32),
            pltpu.VMEM((QROWS,), jnp.float32),
            pltpu.VMEM((MM,), jnp.float32),
            pltpu.SemaphoreType.DMA,
            pltpu.SemaphoreType.DMA,
        ],
    )
    def k(x_hbm, mf_hbm, out_hbm, buf0, buf1, mfb, acc, sem0, sem1):
        wid = lax.axis_index("s") * NC + lax.axis_index("c")

        def start(ch_row0, buf, sem):
            pltpu.make_async_copy(
                x_hbm.at[pl.ds(ch_row0, CH)], buf, sem
            ).start()

        def wait(buf, sem):
            pltpu.make_async_copy(x_hbm.at[pl.ds(0, CH)], buf, sem).wait()

        def compute(buf, moff):
            def row_body(rg, _):
                r0 = rg * VL
                mvec = mfb[pl.ds(moff + r0, VL)]
                ms = [mvec[j] for j in range(VL)]

                def col_body(cb, _):
                    for cs in range(COL_U):
                        o = cb * (COL_U * VL) + cs * VL
                        v = acc[pl.ds(o, VL)]
                        for j in range(VL):
                            v = v + buf[r0 + j, pl.ds(o, VL)] * ms[j]
                        acc[pl.ds(o, VL)] = v
                    return 0

                lax.fori_loop(0, MM // (COL_U * VL), col_body, 0)
                return 0

            lax.fori_loop(0, CH // VL, row_body, 0)

        def task_body(ti, _):
            t = wid * TPW + ti
            seg = t // NQ
            q = t - seg * NQ
            row0 = t * QROWS
            pltpu.sync_copy(mf_hbm.at[t], mfb)

            def zero_body(c, _):
                acc[pl.ds(c * VL, VL)] = jnp.zeros((VL,), jnp.float32)
                return 0

            lax.fori_loop(0, MM // VL, zero_body, 0)

            start(row0, buf0, sem0)

            def pair_body(cp, _):
                c0 = cp * 2
                start(row0 + (c0 + 1) * CH, buf1, sem1)
                wait(buf0, sem0)
                compute(buf0, c0 * CH)

                @pl.when(c0 + 2 < NCH)
                def _():
                    start(row0 + (c0 + 2) * CH, buf0, sem0)

                wait(buf1, sem1)
                compute(buf1, (c0 + 1) * CH)
                return 0

            lax.fori_loop(0, NCH // 2, pair_body, 0)
            pltpu.sync_copy(acc, out_hbm.at[q, seg])
            return 0

        lax.fori_loop(0, TPW, task_body, 0)

    return k(x2, mf96)


def _tc_finish(part, mf24, gf_col, w_proj, b_proj, wc24, bc24):
    def body(part_ref, mf_ref, gf_ref, wp_ref, bp_ref, wc_ref, bc_ref, out_ref):
        seg_sum = part_ref[0] + part_ref[1] + part_ref[2] + part_ref[3]
        den = jnp.sum(mf_ref[:], axis=1, keepdims=True)
        logits = lax.dot_general(
            wc_ref[:], gf_ref[:], (((1,), (0,)), ((), ())),
            precision=lax.Precision.HIGHEST,
        ) + bc_ref[:]
        m = jnp.max(logits, axis=0, keepdims=True)
        e = jnp.exp(logits - m)
        s = jnp.sum(e, axis=0, keepdims=True) / B
        w24 = e / s
        scaled = seg_sum * (w24 / den)
        ri = lax.broadcasted_iota(jnp.int32, (B, NSEG), 0)
        cj = lax.broadcasted_iota(jnp.int32, (B, NSEG), 1)
        sel = jnp.where((cj >= S * ri) & (cj < S * ri + S), 1.0, 0.0)
        xw = lax.dot_general(
            sel, scaled, (((1,), (0,)), ((), ())),
            precision=lax.Precision.HIGHEST,
        )
        out_ref[:] = lax.dot_general(
            xw, wp_ref[:], (((1,), (1,)), ((), ())),
            precision=lax.Precision.HIGHEST,
        ) + bp_ref[:]

    return pl.pallas_call(
        body, out_shape=jax.ShapeDtypeStruct((B, H), jnp.float32)
    )(part, mf24, gf_col, w_proj, b_proj, wc24, bc24)


@jax.jit
def kernel(graph_feature, x_tensors, x_mask, W_proj, b_proj, W_comb, b_comb):
    mf = (~x_mask).astype(jnp.float32)
    x2 = x_tensors.reshape(B * S * L, MM)
    mf96 = mf.reshape(NTASK, QROWS)
    part = _sc_partial_sums(x2, mf96)
    mf24 = mf.reshape(NSEG, L)
    gf_col = graph_feature.reshape(MM, 1)
    wc24 = jnp.tile(W_comb, (B, 1))
    bc24 = jnp.tile(b_comb, (B,)).reshape(NSEG, 1)
    bp = b_proj.reshape(1, H)
    return _tc_finish(part, mf24, gf_col, W_proj, bp, wc24, bc24)


# speedup vs baseline: 2.0201x; 1.4004x over previous
import functools

import jax
import jax.numpy as jnp
from jax import lax
from jax.experimental import pallas as pl
from jax.experimental.pallas import tpu as pltpu
from jax.experimental.pallas import tpu_sc as plsc

B, S, L, MM, H = 8, 3, 2048, 1024, 1024
NC, NS = 2, 16
NW = NC * NS
NSEG = B * S
VL = 16
COL_U = 4

RSC = 512
NQ = 4
NTASK = NSEG * NQ
TPW = NTASK // NW
RPT = RSC // NQ
CH = 32
NCH = RPT // CH

LTC = L - RSC
TCB = 512
NLB = LTC // TCB


def _sc_partial_sums(x2, mf_sc):
    mesh = plsc.VectorSubcoreMesh(
        core_axis_name="c", subcore_axis_name="s", num_cores=NC, num_subcores=NS
    )

    @functools.partial(
        pl.kernel,
        out_type=jax.ShapeDtypeStruct((NQ, NSEG, MM), jnp.float32),
        mesh=mesh,
        scratch_types=[
            pltpu.VMEM((CH, MM), jnp.float32),
            pltpu.VMEM((CH, MM), jnp.float32),
            pltpu.VMEM((RPT,), jnp.float32),
            pltpu.VMEM((MM,), jnp.float32),
            pltpu.SemaphoreType.DMA,
            pltpu.SemaphoreType.DMA,
        ],
    )
    def k(x_hbm, mf_hbm, out_hbm, buf0, buf1, mfb, acc, sem0, sem1):
        wid = lax.axis_index("s") * NC + lax.axis_index("c")

        def start(ch_row0, buf, sem):
            pltpu.make_async_copy(
                x_hbm.at[pl.ds(ch_row0, CH)], buf, sem
            ).start()

        def wait(buf, sem):
            pltpu.make_async_copy(x_hbm.at[pl.ds(0, CH)], buf, sem).wait()

        def compute(buf, moff):
            def row_body(rg, _):
                r0 = rg * VL
                mvec = mfb[pl.ds(moff + r0, VL)]
                ms = [mvec[j] for j in range(VL)]

                def col_body(cb, _):
                    for cs in range(COL_U):
                        o = cb * (COL_U * VL) + cs * VL
                        v = acc[pl.ds(o, VL)]
                        for j in range(VL):
                            v = v + buf[r0 + j, pl.ds(o, VL)] * ms[j]
                        acc[pl.ds(o, VL)] = v
                    return 0

                lax.fori_loop(0, MM // (COL_U * VL), col_body, 0)
                return 0

            lax.fori_loop(0, CH // VL, row_body, 0)

        def task_body(ti, _):
            t = wid * TPW + ti
            seg = t // NQ
            q = t - seg * NQ
            row0 = seg * L + q * RPT
            pltpu.sync_copy(mf_hbm.at[t], mfb)

            def zero_body(c, _):
                acc[pl.ds(c * VL, VL)] = jnp.zeros((VL,), jnp.float32)
                return 0

            lax.fori_loop(0, MM // VL, zero_body, 0)

            start(row0, buf0, sem0)

            def pair_body(cp, _):
                c0 = cp * 2
                start(row0 + (c0 + 1) * CH, buf1, sem1)
                wait(buf0, sem0)
                compute(buf0, c0 * CH)

                @pl.when(c0 + 2 < NCH)
                def _():
                    start(row0 + (c0 + 2) * CH, buf0, sem0)

                wait(buf1, sem1)
                compute(buf1, (c0 + 1) * CH)
                return 0

            lax.fori_loop(0, NCH // 2, pair_body, 0)
            pltpu.sync_copy(acc, out_hbm.at[q, seg])
            return 0

        lax.fori_loop(0, TPW, task_body, 0)

    return k(x2, mf_sc)


def _tc_partial_sums(x24, mf_lb):
    nb_per_seg = L // TCB
    skip = RSC // TCB

    def body(mf_ref, x_ref, out_ref):
        lb = pl.program_id(1)

        @pl.when(lb == 0)
        def _():
            out_ref[...] = jnp.zeros_like(out_ref)

        out_ref[...] += lax.dot_general(
            mf_ref[0], x_ref[0], (((1,), (0,)), ((), ())),
            precision=lax.Precision.HIGHEST,
        )

    return pl.pallas_call(
        body,
        grid=(NSEG, NLB),
        in_specs=[
            pl.BlockSpec(
                (1, 1, TCB),
                lambda s_i, l_i: (s_i * nb_per_seg + l_i + skip, 0, 0),
            ),
            pl.BlockSpec((1, TCB, MM), lambda s_i, l_i: (s_i, l_i + skip, 0)),
        ],
        out_specs=pl.BlockSpec((1, 1, MM), lambda s_i, l_i: (s_i, 0, 0)),
        out_shape=jax.ShapeDtypeStruct((NSEG, 1, MM), jnp.float32),
    )(mf_lb, x24)


def _tc_finish(part_sc, part_tc, mf24, gf_col, w_proj, b_proj, wc24, bc24):
    def body(psc_ref, ptc_ref, mf_ref, gf_ref, wp_ref, bp_ref, wc_ref, bc_ref,
             out_ref):
        seg_sum = (psc_ref[0] + psc_ref[1] + psc_ref[2] + psc_ref[3]
                   + ptc_ref[:, 0])
        den = jnp.sum(mf_ref[:], axis=1, keepdims=True)
        logits = lax.dot_general(
            wc_ref[:], gf_ref[:], (((1,), (0,)), ((), ())),
            precision=lax.Precision.HIGHEST,
        ) + bc_ref[:]
        m = jnp.max(logits, axis=0, keepdims=True)
        e = jnp.exp(logits - m)
        s = jnp.sum(e, axis=0, keepdims=True) / B
        w24 = e / s
        scaled = seg_sum * (w24 / den)
        ri = lax.broadcasted_iota(jnp.int32, (B, NSEG), 0)
        cj = lax.broadcasted_iota(jnp.int32, (B, NSEG), 1)
        sel = jnp.where((cj >= S * ri) & (cj < S * ri + S), 1.0, 0.0)
        xw = lax.dot_general(
            sel, scaled, (((1,), (0,)), ((), ())),
            precision=lax.Precision.HIGHEST,
        )
        out_ref[:] = lax.dot_general(
            xw, wp_ref[:], (((1,), (1,)), ((), ())),
            precision=lax.Precision.HIGHEST,
        ) + bp_ref[:]

    return pl.pallas_call(
        body, out_shape=jax.ShapeDtypeStruct((B, H), jnp.float32)
    )(part_sc, part_tc, mf24, gf_col, w_proj, b_proj, wc24, bc24)


@jax.jit
def kernel(graph_feature, x_tensors, x_mask, W_proj, b_proj, W_comb, b_comb):
    mf = (~x_mask).astype(jnp.float32)
    x2 = x_tensors.reshape(B * S * L, MM)
    x24 = x_tensors.reshape(NSEG, L, MM)
    mf24 = mf.reshape(NSEG, L)
    mf_sc = mf24[:, :RSC].reshape(NTASK, RPT)
    part_sc = _sc_partial_sums(x2, mf_sc)
    mf_lb = mf.reshape(NSEG * (L // TCB), 1, TCB)
    part_tc = _tc_partial_sums(x24, mf_lb)
    gf_col = graph_feature.reshape(MM, 1)
    wc24 = jnp.tile(W_comb, (B, 1))
    bc24 = jnp.tile(b_comb, (B,)).reshape(NSEG, 1)
    bp = b_proj.reshape(1, H)
    return _tc_finish(part_sc, part_tc, mf24, gf_col, W_proj, bp, wc24, bc24)


# speedup vs baseline: 2.0222x; 1.0010x over previous
import functools

import jax
import jax.numpy as jnp
from jax import lax
from jax.experimental import pallas as pl
from jax.experimental.pallas import tpu as pltpu
from jax.experimental.pallas import tpu_sc as plsc

B, S, L, MM, H = 8, 3, 2048, 1024, 1024
NC, NS = 2, 16
NW = NC * NS
NSEG = B * S
VL = 16
COL_U = 4

RSC = 512
NQ = 4
NTASK = NSEG * NQ
TPW = NTASK // NW
RPT = RSC // NQ
CH = 32
NCH = RPT // CH

LTC = L - RSC
TCB = 512
NLB = LTC // TCB


def _sc_partial_sums(x2, mf_sc):
    mesh = plsc.VectorSubcoreMesh(
        core_axis_name="c", subcore_axis_name="s", num_cores=NC, num_subcores=NS
    )

    @functools.partial(
        pl.kernel,
        out_type=jax.ShapeDtypeStruct((NQ, NSEG, MM), jnp.float32),
        mesh=mesh,
        scratch_types=[
            pltpu.VMEM((CH, MM), jnp.float32),
            pltpu.VMEM((CH, MM), jnp.float32),
            pltpu.VMEM((RPT,), jnp.float32),
            pltpu.VMEM((MM,), jnp.float32),
            pltpu.SemaphoreType.DMA,
            pltpu.SemaphoreType.DMA,
        ],
    )
    def k(x_hbm, mf_hbm, out_hbm, buf0, buf1, mfb, acc, sem0, sem1):
        wid = lax.axis_index("s") * NC + lax.axis_index("c")

        def start(ch_row0, buf, sem):
            pltpu.make_async_copy(
                x_hbm.at[pl.ds(ch_row0, CH)], buf, sem
            ).start()

        def wait(buf, sem):
            pltpu.make_async_copy(x_hbm.at[pl.ds(0, CH)], buf, sem).wait()

        def compute(buf, moff):
            def row_body(rg, _):
                r0 = rg * VL
                mvec = mfb[pl.ds(moff + r0, VL)]
                ms = [mvec[j] for j in range(VL)]

                def col_body(cb, _):
                    for cs in range(COL_U):
                        o = cb * (COL_U * VL) + cs * VL
                        v = acc[pl.ds(o, VL)]
                        for j in range(VL):
                            v = v + buf[r0 + j, pl.ds(o, VL)] * ms[j]
                        acc[pl.ds(o, VL)] = v
                    return 0

                lax.fori_loop(0, MM // (COL_U * VL), col_body, 0)
                return 0

            lax.fori_loop(0, CH // VL, row_body, 0)

        def task_body(ti, _):
            t = wid * TPW + ti
            seg = t // NQ
            q = t - seg * NQ
            row0 = seg * L + q * RPT
            pltpu.sync_copy(mf_hbm.at[t], mfb)

            def zero_body(c, _):
                acc[pl.ds(c * VL, VL)] = jnp.zeros((VL,), jnp.float32)
                return 0

            lax.fori_loop(0, MM // VL, zero_body, 0)

            start(row0, buf0, sem0)

            def pair_body(cp, _):
                c0 = cp * 2
                start(row0 + (c0 + 1) * CH, buf1, sem1)
                wait(buf0, sem0)
                compute(buf0, c0 * CH)

                @pl.when(c0 + 2 < NCH)
                def _():
                    start(row0 + (c0 + 2) * CH, buf0, sem0)

                wait(buf1, sem1)
                compute(buf1, (c0 + 1) * CH)
                return 0

            lax.fori_loop(0, NCH // 2, pair_body, 0)
            pltpu.sync_copy(acc, out_hbm.at[q, seg])
            return 0

        lax.fori_loop(0, TPW, task_body, 0)

    return k(x2, mf_sc)


def _tc_partial_sums(x24, mf_lb):
    nb_per_seg = L // TCB
    skip = RSC // TCB

    def body(mf_ref, x_ref, out_ref):
        lb = pl.program_id(1)

        @pl.when(lb == 0)
        def _():
            out_ref[...] = jnp.zeros_like(out_ref)

        out_ref[...] += jnp.sum(
            x_ref[0] * mf_ref[0], axis=0, keepdims=True
        )[None]

    return pl.pallas_call(
        body,
        grid=(NSEG, NLB),
        in_specs=[
            pl.BlockSpec(
                (1, TCB, 1),
                lambda s_i, l_i: (s_i * nb_per_seg + l_i + skip, 0, 0),
            ),
            pl.BlockSpec((1, TCB, MM), lambda s_i, l_i: (s_i, l_i + skip, 0)),
        ],
        out_specs=pl.BlockSpec((1, 1, MM), lambda s_i, l_i: (s_i, 0, 0)),
        out_shape=jax.ShapeDtypeStruct((NSEG, 1, MM), jnp.float32),
    )(mf_lb, x24)


def _tc_finish(part_sc, part_tc, mf24, gf_col, w_proj, b_proj, wc24, bc24):
    def body(psc_ref, ptc_ref, mf_ref, gf_ref, wp_ref, bp_ref, wc_ref, bc_ref,
             out_ref):
        seg_sum = (psc_ref[0] + psc_ref[1] + psc_ref[2] + psc_ref[3]
                   + ptc_ref[:, 0])
        den = jnp.sum(mf_ref[:], axis=1, keepdims=True)
        logits = lax.dot_general(
            wc_ref[:], gf_ref[:], (((1,), (0,)), ((), ())),
            precision=lax.Precision.HIGHEST,
        ) + bc_ref[:]
        m = jnp.max(logits, axis=0, keepdims=True)
        e = jnp.exp(logits - m)
        s = jnp.sum(e, axis=0, keepdims=True) / B
        w24 = e / s
        scaled = seg_sum * (w24 / den)
        ri = lax.broadcasted_iota(jnp.int32, (B, NSEG), 0)
        cj = lax.broadcasted_iota(jnp.int32, (B, NSEG), 1)
        sel = jnp.where((cj >= S * ri) & (cj < S * ri + S), 1.0, 0.0)
        xw = lax.dot_general(
            sel, scaled, (((1,), (0,)), ((), ())),
            precision=lax.Precision.HIGHEST,
        )
        out_ref[:] = lax.dot_general(
            xw, wp_ref[:], (((1,), (1,)), ((), ())),
            precision=lax.Precision.HIGHEST,
        ) + bp_ref[:]

    return pl.pallas_call(
        body, out_shape=jax.ShapeDtypeStruct((B, H), jnp.float32)
    )(part_sc, part_tc, mf24, gf_col, w_proj, b_proj, wc24, bc24)


@jax.jit
def kernel(graph_feature, x_tensors, x_mask, W_proj, b_proj, W_comb, b_comb):
    mf = (~x_mask).astype(jnp.float32)
    x2 = x_tensors.reshape(B * S * L, MM)
    x24 = x_tensors.reshape(NSEG, L, MM)
    mf24 = mf.reshape(NSEG, L)
    mf_sc = mf24[:, :RSC].reshape(NTASK, RPT)
    part_sc = _sc_partial_sums(x2, mf_sc)
    mf_lb = mf.reshape(NSEG * (L // TCB), TCB, 1)
    part_tc = _tc_partial_sums(x24, mf_lb)
    gf_col = graph_feature.reshape(MM, 1)
    wc24 = jnp.tile(W_comb, (B, 1))
    bc24 = jnp.tile(b_comb, (B,)).reshape(NSEG, 1)
    bp = b_proj.reshape(1, H)
    return _tc_finish(part_sc, part_tc, mf24, gf_col, W_proj, bp, wc24, bc24)
